# baseline (speedup 1.0000x reference)
import functools

import jax
import jax.numpy as jnp
from jax import lax
from jax.experimental import pallas as pl
from jax.experimental.pallas import tpu as pltpu
from jax.experimental.pallas import tpu_sc as plsc

NINP = 128
D1 = 32
D2 = 8
N_TOK = 16384
C1 = 20000
C2 = 200000
C3 = 1000000
NT0 = C2 - C1
NT1 = C3 - C2

NC = 2
NS = 16
NW = NC * NS
BPW = N_TOK // NW
L = 16
GCH = 128
NCH = BPW // GCH
NE1 = BPW * D2
NCH1 = NE1 // GCH
DUMMY_MASK = 0x3FFF

_SC_PARAMS = dict(
    mesh=plsc.VectorSubcoreMesh(core_axis_name="c", subcore_axis_name="s"),
    compiler_params=pltpu.CompilerParams(
        use_tc_tiling_on_sc=False, needs_layout_passes=False),
)


def _stage_a1_body(ids_hbm, head_hbm, t1f_hbm, o_out,
                   ids_v, hidx_v, i1e_v, o_v, g1f_v, sem):
    wid = lax.axis_index("s") * NC + lax.axis_index("c")
    base = wid * BPW

    pltpu.sync_copy(ids_hbm.at[pl.ds(base, BPW)], ids_v)
    lanes = lax.iota(jnp.int32, L)

    for i in range(BPW // L):
        v = ids_v[pl.ds(i * L, L)]
        spread = v & DUMMY_MASK
        hidx = jnp.where(v < C1, v, spread)
        r1 = jnp.where(v >= C2, v - C2, spread)
        e1 = ((r1 >> 7) << 10) + (r1 & 127)
        r, c = i // (GCH // L), (i % (GCH // L)) * L
        hidx_v[r, pl.ds(c, L)] = hidx
        pdst = (lanes + i * L) * D2
        for k in range(D2):
            plsc.store_scatter(i1e_v, [pdst + k], e1 + (k << 7))

    copies = []
    for ch in range(NCH):
        sl = pl.ds(ch * GCH, GCH)
        copies.append(pltpu.async_copy(head_hbm.at[hidx_v.at[ch]], o_v.at[sl], sem))
    for ch in range(NCH1):
        sl = pl.ds(ch * GCH, GCH)
        copies.append(pltpu.async_copy(t1f_hbm.at[i1e_v.at[sl]], g1f_v.at[sl], sem))
    for cp in copies:
        cp.wait()

    for g in range(BPW // L):
        toks = lanes + g * L
        v = ids_v[pl.ds(g * L, L)]
        in1 = v >= C2
        t8 = toks * D2
        for k in range(D2):
            val = plsc.load_gather(g1f_v, [t8 + k])
            kk = jnp.full((L,), D1 + k, jnp.int32)
            plsc.store_scatter(o_v, [toks, kk], val, mask=in1)

    pltpu.sync_copy(o_v, o_out.at[pl.ds(base, BPW)])


_stage_a1 = functools.partial(
    pl.kernel,
    out_type=jax.ShapeDtypeStruct((N_TOK, NINP), jnp.float32),
    scratch_types=[
        pltpu.VMEM((BPW,), jnp.int32),
        pltpu.VMEM((NCH, GCH), jnp.int32),
        pltpu.VMEM((NE1,), jnp.int32),
        pltpu.VMEM((BPW, NINP), jnp.float32),
        pltpu.VMEM((NE1,), jnp.float32),
        pltpu.SemaphoreType.DMA,
    ],
    **_SC_PARAMS,
)(_stage_a1_body)


NE0 = BPW * D1
NCH0 = NE0 // GCH
WAVE = 32


def _stage_a2_body(ids_hbm, t0f_hbm, g0_out, ids_v, i0e_v, g0f_v, sem):
    wid = lax.axis_index("s") * NC + lax.axis_index("c")
    base = wid * BPW

    pltpu.sync_copy(ids_hbm.at[pl.ds(base, BPW)], ids_v)
    lanes = lax.iota(jnp.int32, L)

    for i in range(BPW // L):
        v = ids_v[pl.ds(i * L, L)]
        in0 = (v >= C1) & (v < C2)
        i0 = jnp.where(in0, v - C1, v & DUMMY_MASK)
        pdst = (lanes + i * L) * D1
        for k in range(D1):
            plsc.store_scatter(i0e_v, [pdst + k], i0 + k * NT0)

    for w in range(NCH0 // WAVE):
        copies = []
        for ch in range(w * WAVE, (w + 1) * WAVE):
            sl = pl.ds(ch * GCH, GCH)
            copies.append(pltpu.async_copy(t0f_hbm.at[i0e_v.at[sl]], g0f_v.at[sl], sem))
        for cp in copies:
            cp.wait()

    pltpu.sync_copy(g0f_v, g0_out.at[pl.ds(base * D1, NE0)])


_stage_a2 = functools.partial(
    pl.kernel,
    out_type=jax.ShapeDtypeStruct((N_TOK * D1,), jnp.float32),
    scratch_types=[
        pltpu.VMEM((BPW,), jnp.int32),
        pltpu.VMEM((NE0,), jnp.int32),
        pltpu.VMEM((NE0,), jnp.float32),
        pltpu.SemaphoreType.DMA,
    ],
    **_SC_PARAMS,
)(_stage_a2_body)


TB = 4096


def _stage_b_body(ids_ref, o_ref, g0_ref, p0t_ref, p1c_ref, out_ref):
    ids = ids_ref[...]
    o = o_ref[...]
    y0 = jnp.dot(g0_ref[...], p0t_ref[...], preferred_element_type=jnp.float32)
    y1 = jnp.dot(o, p1c_ref[...], preferred_element_type=jnp.float32)
    out_ref[...] = jnp.where(ids < C1, o, jnp.where(ids < C2, y0, y1))


def kernel(input, head_w, tail0_emb, tail0_proj, tail1_emb, tail1_proj):
    t1flat = tail1_emb.reshape(NT1 // 128, 128, D2).swapaxes(1, 2).reshape(-1)
    t0fm = jax.lax.optimization_barrier(tail0_emb.T.reshape(-1))
    o = _stage_a1(input, head_w, t1flat)
    g0 = _stage_a2(input, t0fm).reshape(N_TOK, D1)
    ids2d = input.reshape(N_TOK, 1)
    p0t = tail0_proj.T
    p1c = jnp.zeros((NINP, NINP), jnp.float32).at[D1:D1 + D2, :].set(tail1_proj.T)
    out = pl.pallas_call(
        _stage_b_body,
        grid=(N_TOK // TB,),
        in_specs=[
            pl.BlockSpec((TB, 1), lambda i: (i, 0)),
            pl.BlockSpec((TB, NINP), lambda i: (i, 0)),
            pl.BlockSpec((TB, D1), lambda i: (i, 0)),
            pl.BlockSpec((D1, NINP), lambda i: (0, 0)),
            pl.BlockSpec((NINP, NINP), lambda i: (0, 0)),
        ],
        out_specs=pl.BlockSpec((TB, NINP), lambda i: (i, 0)),
        out_shape=jax.ShapeDtypeStruct((N_TOK, NINP), jnp.float32),
    )(ids2d, o, g0, p0t, p1c)
    return out


# speedup vs baseline: 2.3346x; 2.3346x over previous
import functools

import jax
import jax.numpy as jnp
from jax import lax
from jax.experimental import pallas as pl
from jax.experimental.pallas import tpu as pltpu
from jax.experimental.pallas import tpu_sc as plsc

NINP = 128
D1 = 32
D2 = 8
N_TOK = 16384
C1 = 20000
C2 = 200000
C3 = 1000000
NT0 = C2 - C1
NT1 = C3 - C2

NC = 2
NS = 16
NW = NC * NS
BPW = N_TOK // NW
L = 16
GCH = 128
NCH = BPW // GCH
NE1 = BPW * D2
NCH1 = NE1 // GCH
DUMMY_MASK = 0x3FFF

_SC_PARAMS = dict(
    mesh=plsc.VectorSubcoreMesh(core_axis_name="c", subcore_axis_name="s"),
    compiler_params=pltpu.CompilerParams(
        use_tc_tiling_on_sc=False, needs_layout_passes=False),
)


def _stage_a1_body(ids_hbm, head_hbm, t1f_hbm, o_out,
                   ids_v, hidx_v, i1e_v, o_v, g1f_v, sem):
    wid = lax.axis_index("s") * NC + lax.axis_index("c")
    base = wid * BPW

    pltpu.sync_copy(ids_hbm.at[pl.ds(base, BPW)], ids_v)
    lanes = lax.iota(jnp.int32, L)

    for i in range(BPW // L):
        v = ids_v[pl.ds(i * L, L)]
        spread = v & DUMMY_MASK
        hidx = jnp.where(v < C1, v, spread)
        r1 = jnp.where(v >= C2, v - C2, spread)
        e1 = ((r1 >> 7) << 10) + (r1 & 127)
        r, c = i // (GCH // L), (i % (GCH // L)) * L
        hidx_v[r, pl.ds(c, L)] = hidx
        pdst = (lanes + i * L) * D2
        for k in range(D2):
            plsc.store_scatter(i1e_v, [pdst + k], e1 + (k << 7))

    copies = []
    for ch in range(NCH):
        sl = pl.ds(ch * GCH, GCH)
        copies.append(pltpu.async_copy(head_hbm.at[hidx_v.at[ch]], o_v.at[sl], sem))
    for ch in range(NCH1):
        sl = pl.ds(ch * GCH, GCH)
        copies.append(pltpu.async_copy(t1f_hbm.at[i1e_v.at[sl]], g1f_v.at[sl], sem))
    for cp in copies:
        cp.wait()

    for g in range(BPW // L):
        toks = lanes + g * L
        v = ids_v[pl.ds(g * L, L)]
        in1 = v >= C2
        t8 = toks * D2
        for k in range(D2):
            val = plsc.load_gather(g1f_v, [t8 + k])
            kk = jnp.full((L,), D1 + k, jnp.int32)
            plsc.store_scatter(o_v, [toks, kk], val, mask=in1)

    pltpu.sync_copy(o_v, o_out.at[pl.ds(base, BPW)])


_stage_a1 = functools.partial(
    pl.kernel,
    out_type=jax.ShapeDtypeStruct((N_TOK, NINP), jnp.float32),
    scratch_types=[
        pltpu.VMEM((BPW,), jnp.int32),
        pltpu.VMEM((NCH, GCH), jnp.int32),
        pltpu.VMEM((NE1,), jnp.int32),
        pltpu.VMEM((BPW, NINP), jnp.float32),
        pltpu.VMEM((NE1,), jnp.float32),
        pltpu.SemaphoreType.DMA,
    ],
    **_SC_PARAMS,
)(_stage_a1_body)


def _stage_a2_body(ids_hbm, t0_hbm, g0_out, ids_v, i0_v, g0_v, sem):
    wid = lax.axis_index("s") * NC + lax.axis_index("c")
    base = wid * BPW

    pltpu.sync_copy(ids_hbm.at[pl.ds(base, BPW)], ids_v)
    for i in range(BPW // L):
        v = ids_v[pl.ds(i * L, L)]
        in0 = (v >= C1) & (v < C2)
        i0 = jnp.where(in0, v - C1, v & DUMMY_MASK)
        r, c = i // (GCH // L), (i % (GCH // L)) * L
        i0_v[r, pl.ds(c, L)] = i0

    copies = []
    for ch in range(NCH):
        sl = pl.ds(ch * GCH, GCH)
        copies.append(pltpu.async_copy(t0_hbm.at[i0_v.at[ch]], g0_v.at[sl], sem))
    for cp in copies:
        cp.wait()

    pltpu.sync_copy(g0_v, g0_out.at[pl.ds(base, BPW)])


_stage_a2 = functools.partial(
    pl.kernel,
    out_type=jax.ShapeDtypeStruct((N_TOK, D1), jnp.float32),
    scratch_types=[
        pltpu.VMEM((BPW,), jnp.int32),
        pltpu.VMEM((NCH, GCH), jnp.int32),
        pltpu.VMEM((BPW, D1), jnp.float32),
        pltpu.SemaphoreType.DMA,
    ],
    **_SC_PARAMS,
)(_stage_a2_body)


TB = 4096


def _stage_b_body(ids_ref, o_ref, g0_ref, p0t_ref, p1c_ref, out_ref):
    ids = ids_ref[...]
    o = o_ref[...]
    y0 = jnp.dot(g0_ref[...], p0t_ref[...], preferred_element_type=jnp.float32)
    y1 = jnp.dot(o, p1c_ref[...], preferred_element_type=jnp.float32)
    out_ref[...] = jnp.where(ids < C1, o, jnp.where(ids < C2, y0, y1))


def kernel(input, head_w, tail0_emb, tail0_proj, tail1_emb, tail1_proj):
    t1flat = tail1_emb.reshape(NT1 // 128, 128, D2).swapaxes(1, 2).reshape(-1)
    t0rm = jax.lax.optimization_barrier(tail0_emb.reshape(-1)).reshape(NT0, D1)
    o = _stage_a1(input, head_w, t1flat)
    g0 = _stage_a2(input, t0rm)
    ids2d = input.reshape(N_TOK, 1)
    p0t = tail0_proj.T
    p1c = jnp.zeros((NINP, NINP), jnp.float32).at[D1:D1 + D2, :].set(tail1_proj.T)
    out = pl.pallas_call(
        _stage_b_body,
        grid=(N_TOK // TB,),
        in_specs=[
            pl.BlockSpec((TB, 1), lambda i: (i, 0)),
            pl.BlockSpec((TB, NINP), lambda i: (i, 0)),
            pl.BlockSpec((TB, D1), lambda i: (i, 0)),
            pl.BlockSpec((D1, NINP), lambda i: (0, 0)),
            pl.BlockSpec((NINP, NINP), lambda i: (0, 0)),
        ],
        out_specs=pl.BlockSpec((TB, NINP), lambda i: (i, 0)),
        out_shape=jax.ShapeDtypeStruct((N_TOK, NINP), jnp.float32),
    )(ids2d, o, g0, p0t, p1c)
    return out
